# Initial kernel scaffold; baseline (speedup 1.0000x reference)
import jax
import jax.numpy as jnp
from jax.experimental import pallas as pl


def kernel(x, pos_table):
    raise NotImplementedError("write your pallas kernel here")


# speedup vs baseline: 1.4999x; 1.4999x over previous
import jax
import jax.numpy as jnp
from jax.experimental import pallas as pl


def _add_kernel(x_ref, pos_ref, o_ref):
    o_ref[...] = x_ref[...] + pos_ref[...]


def kernel(x, pos_table):
    B, S, E = x.shape
    S_BLK = 512
    grid = (S // S_BLK, B)
    return pl.pallas_call(
        _add_kernel,
        grid=grid,
        in_specs=[
            pl.BlockSpec((1, S_BLK, E), lambda i, b: (b, i, 0)),
            pl.BlockSpec((S_BLK, E), lambda i, b: (i, 0)),
        ],
        out_specs=pl.BlockSpec((1, S_BLK, E), lambda i, b: (b, i, 0)),
        out_shape=jax.ShapeDtypeStruct((B, S, E), x.dtype),
    )(x, pos_table[:S])


# speedup vs baseline: 1.6698x; 1.1132x over previous
import jax
import jax.numpy as jnp
from jax.experimental import pallas as pl


def _add_kernel(x_ref, pos_ref, o_ref):
    o_ref[...] = x_ref[...] + pos_ref[...]


def kernel(x, pos_table):
    B, S, E = x.shape
    S_BLK = 1024
    grid = (S // S_BLK, B)
    return pl.pallas_call(
        _add_kernel,
        grid=grid,
        in_specs=[
            pl.BlockSpec((1, S_BLK, E), lambda i, b: (b, i, 0)),
            pl.BlockSpec((S_BLK, E), lambda i, b: (i, 0)),
        ],
        out_specs=pl.BlockSpec((1, S_BLK, E), lambda i, b: (b, i, 0)),
        out_shape=jax.ShapeDtypeStruct((B, S, E), x.dtype),
    )(x, pos_table[:S])


# speedup vs baseline: 1.7332x; 1.0380x over previous
import jax
import jax.numpy as jnp
from jax.experimental import pallas as pl


def _add_kernel(x_ref, pos_ref, o_ref):
    o_ref[...] = x_ref[...] + pos_ref[...]


def kernel(x, pos_table):
    B, S, E = x.shape
    S_BLK = 2048
    grid = (S // S_BLK, B)
    return pl.pallas_call(
        _add_kernel,
        grid=grid,
        in_specs=[
            pl.BlockSpec((1, S_BLK, E), lambda i, b: (b, i, 0)),
            pl.BlockSpec((S_BLK, E), lambda i, b: (i, 0)),
        ],
        out_specs=pl.BlockSpec((1, S_BLK, E), lambda i, b: (b, i, 0)),
        out_shape=jax.ShapeDtypeStruct((B, S, E), x.dtype),
    )(x, pos_table[:S])
